# baseline (speedup 1.0000x reference)
import functools

import jax
import jax.numpy as jnp
from jax import lax
from jax.experimental import pallas as pl
from jax.experimental.pallas import tpu as pltpu
from jax.experimental.pallas import tpu_sc as plsc

EMB = 128
NV = 10000
KNN = 11
SIZE = 296
NPIX = SIZE * SIZE
CHUNK = 2744
GRID_ROWS = 32 * CHUNK
HALF = 16 * CHUNK
TRASH0 = NPIX
TRASH1 = NPIX + 8
NQ = 4096
NS = 2048
QBLK = 512
ROWS = 8
NSTRIP = SIZE // ROWS
SPAT = ROWS * SIZE


def _sc_gather_body(emb_hbm, tp8_hbm, vis_hbm, tof_hbm, xyz_hbm,
                    idx_v, emb_v, xyz_v, sem):
    c = lax.axis_index("c")
    s = lax.axis_index("s")
    wid = c * 16 + s
    base = wid * 64
    pltpu.sync_copy(vis_hbm.at[pl.ds(base, 64)], idx_v)
    pltpu.async_copy(emb_hbm.at[idx_v], emb_v, sem).wait()
    pltpu.sync_copy(emb_v, tof_hbm.at[pl.ds(base, 64)])
    pltpu.async_copy(tp8_hbm.at[idx_v], xyz_v, sem).wait()
    pltpu.sync_copy(xyz_v, xyz_hbm.at[pl.ds(base, 64)])


def _sc_gather(emb, tp8, vis):
    mesh = plsc.VectorSubcoreMesh(core_axis_name="c", subcore_axis_name="s")
    kern = functools.partial(
        pl.kernel,
        mesh=mesh,
        compiler_params=pltpu.CompilerParams(use_tc_tiling_on_sc=True),
        out_type=[
            jax.ShapeDtypeStruct((NS, EMB), jnp.float32),
            jax.ShapeDtypeStruct((NS, EMB), jnp.float32),
        ],
        scratch_types=[
            pltpu.VMEM((64,), jnp.int32),
            pltpu.VMEM((64, EMB), jnp.float32),
            pltpu.VMEM((64, EMB), jnp.float32),
            pltpu.SemaphoreType.DMA,
        ],
    )(_sc_gather_body)
    return kern(emb, tp8, vis)


def _sc_scatter_body(interp_hbm, tofp_hbm, fillrow_hbm, fillblk_hbm,
                     aidx0_hbm, aidx1_hbm, pidx0_hbm, pidx1_hbm, grid_hbm,
                     fill_v, idxa_v, rowsa_v, idxp_v, rowsp_v, sem):
    c = lax.axis_index("c")
    s = lax.axis_index("s")
    pltpu.sync_copy(fillblk_hbm, fill_v)
    fbase = (c * 16 + s) * CHUNK
    copies = []
    for j in range(CHUNK // 128):
        copies.append(pltpu.async_copy(
            fill_v, grid_hbm.at[pl.ds(fbase + j * 128, 128)], sem))
    rem = CHUNK - 128 * (CHUNK // 128)
    copies.append(pltpu.async_copy(
        fill_v.at[pl.ds(0, rem)],
        grid_hbm.at[pl.ds(fbase + 128 * (CHUNK // 128), rem)], sem))
    for cp in copies:
        cp.wait()
    plsc.subcore_barrier()
    abase = s * 256

    @pl.when(c == 0)
    def _():
        for j in range(2):
            pltpu.sync_copy(aidx0_hbm.at[pl.ds(abase + j * 128, 128)],
                            idxa_v.at[j])

    @pl.when(c == 1)
    def _():
        for j in range(2):
            pltpu.sync_copy(aidx1_hbm.at[pl.ds(abase + j * 128, 128)],
                            idxa_v.at[j])

    for j in range(2):
        pltpu.sync_copy(interp_hbm.at[pl.ds(abase + j * 128, 128)],
                        rowsa_v.at[j])
        pltpu.async_copy(rowsa_v.at[j], grid_hbm.at[idxa_v.at[j]], sem).wait()
    plsc.subcore_barrier()
    pbase = s * 128

    @pl.when(c == 0)
    def _():
        pltpu.sync_copy(pidx0_hbm.at[pl.ds(pbase, 128)], idxp_v)

    @pl.when(c == 1)
    def _():
        pltpu.sync_copy(pidx1_hbm.at[pl.ds(pbase, 128)], idxp_v)

    pltpu.sync_copy(tofp_hbm.at[pl.ds(pbase, 128)], rowsp_v)
    pltpu.async_copy(rowsp_v, grid_hbm.at[idxp_v], sem).wait()
    plsc.subcore_barrier()
    @pl.when((c == 0) & (s == 0))
    def _():
        pltpu.sync_copy(fillrow_hbm, grid_hbm.at[pl.ds(0, 1)])


def _sc_scatter(interp, tofp, fillrow, fillblk, aidx0, aidx1, pidx0, pidx1):
    mesh = plsc.VectorSubcoreMesh(core_axis_name="c", subcore_axis_name="s")
    kern = functools.partial(
        pl.kernel,
        mesh=mesh,
        compiler_params=pltpu.CompilerParams(use_tc_tiling_on_sc=True),
        out_type=jax.ShapeDtypeStruct((GRID_ROWS, EMB), jnp.float32),
        scratch_types=[
            pltpu.VMEM((128, EMB), jnp.float32),
            pltpu.VMEM((2, 128), jnp.int32),
            pltpu.VMEM((2, 128, EMB), jnp.float32),
            pltpu.VMEM((128,), jnp.int32),
            pltpu.VMEM((128, EMB), jnp.float32),
            pltpu.SemaphoreType.DMA,
        ],
    )(_sc_scatter_body)
    return kern(interp, tofp, fillrow, fillblk, aidx0, aidx1, pidx0, pidx1)


def _geom_body(xyz_ref, tr_ref, tof0_ref, fill_ref, apm_ref,
               tofp_ref, pcoord_ref, pidx0_ref, pidx1_ref,
               aidx0_ref, aidx1_ref):
    ph = xyz_ref[:, 0:4]
    tr = tr_ref[...]
    cam = lax.dot_general(ph, tr, (((1,), (1,)), ((), ())),
                          preferred_element_type=jnp.float32)
    x = cam[:, 0]
    y = cam[:, 1]
    z = cam[:, 2]
    zs = jnp.where(jnp.abs(z) < 1e-6, 1e-6, z)
    u = 12.0 * x / zs + SIZE / 2.0
    v = 12.0 * y / zs + SIZE / 2.0
    px = jnp.round(u)
    py = jnp.round(v)
    bad = (px < 0.0) | (px > 295.0) | (py < 0.0) | (py > 295.0)
    px = jnp.where(bad, 0.0, px)
    py = jnp.where(bad, 0.0, py)
    tofp_ref[...] = jnp.where(bad[:, None], fill_ref[...][None, :],
                              tof0_ref[...])
    pcoord_ref[0:1, :] = px.reshape(1, NS)
    pcoord_ref[1:2, :] = py.reshape(1, NS)
    pcoord_ref[2:8, :] = jnp.zeros((6, NS), jnp.float32)
    pid = py.astype(jnp.int32) * SIZE + px.astype(jnp.int32)
    pid_row = pid.reshape(1, NS)
    col = lax.broadcasted_iota(jnp.int32, (1, NS), 1)
    loser_parts = []
    for cblk in range(4):
        pc_chunk = pid[cblk * 512:(cblk + 1) * 512]
        row_ids = lax.broadcasted_iota(jnp.int32, (512, 1), 0) + cblk * 512
        eq = (pc_chunk[:, None] == pid_row) & (col > row_ids)
        loser_parts.append(jnp.any(eq, axis=1).astype(jnp.int32))
    loser = jnp.concatenate(loser_parts, axis=0)
    own0 = pid < HALF
    keep = loser == 0
    pidx0_ref[...] = jnp.where(own0 & keep, pid, TRASH0)
    pidx1_ref[...] = jnp.where(jnp.logical_not(own0) & keep, pid, TRASH1)
    apm = apm_ref[...]
    aid = apm[:, 1] * SIZE + apm[:, 0]
    aown0 = aid < HALF
    aidx0_ref[...] = jnp.where(aown0, aid, TRASH0)
    aidx1_ref[...] = jnp.where(jnp.logical_not(aown0), aid, TRASH1)


def _geom(xyz8, transform, tof0, filler, apm):
    return pl.pallas_call(
        _geom_body,
        out_shape=[
            jax.ShapeDtypeStruct((NS, EMB), jnp.float32),
            jax.ShapeDtypeStruct((8, NS), jnp.float32),
            jax.ShapeDtypeStruct((NS,), jnp.int32),
            jax.ShapeDtypeStruct((NS,), jnp.int32),
            jax.ShapeDtypeStruct((NQ,), jnp.int32),
            jax.ShapeDtypeStruct((NQ,), jnp.int32),
        ],
    )(xyz8, transform, tof0, filler, apm)


def _knn_body(apmf_ref, pcoord_ref, tofp_ref, out_ref):
    qx = apmf_ref[:, 0:1]
    qy = apmf_ref[:, 1:2]
    px = pcoord_ref[0:1, :]
    py = pcoord_ref[1:2, :]
    dx = qx - px
    dy = qy - py
    d2 = dx * dx + dy * dy
    col = lax.broadcasted_iota(jnp.int32, (QBLK, NS), 1)
    key = d2.astype(jnp.int32) * NS + col
    wmat = jnp.zeros((QBLK, NS), jnp.float32)
    wsum = jnp.zeros((QBLK, 1), jnp.float32)
    big = jnp.int32(2147483647)
    for _ in range(KNN):
        m = jnp.min(key, axis=1, keepdims=True)
        sel = key == m
        d2f = (m // NS).astype(jnp.float32)
        wt = 1.0 / (jnp.sqrt(d2f + 1e-12) + 1e-8)
        wmat = wmat + jnp.where(sel, wt, 0.0)
        wsum = wsum + wt
        key = jnp.where(sel, big, key)
    wmat = wmat / wsum
    out_ref[...] = jnp.dot(wmat, tofp_ref[...],
                           preferred_element_type=jnp.float32)


def _knn(apmf, pcoord, tofp):
    return pl.pallas_call(
        _knn_body,
        grid=(NQ // QBLK,),
        in_specs=[
            pl.BlockSpec((QBLK, 2), lambda i: (i, 0)),
            pl.BlockSpec((8, NS), lambda i: (0, 0)),
            pl.BlockSpec((NS, EMB), lambda i: (0, 0)),
        ],
        out_specs=pl.BlockSpec((QBLK, EMB), lambda i: (i, 0)),
        out_shape=jax.ShapeDtypeStruct((NQ, EMB), jnp.float32),
    )(apmf, pcoord, tofp)


def _strip_chw(prev, cur, nxt, i, cin, dtype):
    top = jnp.where(i == 0, jnp.zeros((cin, 1, SIZE), dtype), prev[:, ROWS - 1:ROWS, :])
    bot = jnp.where(i == NSTRIP - 1, jnp.zeros((cin, 1, SIZE), dtype), nxt[:, 0:1, :])
    rows = jnp.concatenate([top, cur, bot], axis=1)
    zc = jnp.zeros((cin, ROWS + 2, 1), dtype)
    return jnp.concatenate([zc, rows, zc], axis=2)


def _conv_taps(strip, w_ref, cin, cout):
    acc = jnp.zeros((cout, SPAT), jnp.float32)
    for kh in range(3):
        for kw in range(3):
            xt = strip[:, kh:kh + ROWS, kw:kw + SIZE].reshape(cin, SPAT)
            acc = acc + lax.dot_general(
                w_ref[kh, kw], xt, (((1,), (0,)), ((), ())),
                preferred_element_type=jnp.float32)
    return acc


def _make_conv_chw(cin, cout, relu, out_dtype):
    def body(prev_ref, cur_ref, nxt_ref, w_ref, a_ref, b_ref, out_ref):
        i = pl.program_id(0)
        strip = _strip_chw(prev_ref[...], cur_ref[...], nxt_ref[...], i,
                           cin, jnp.bfloat16)
        acc = _conv_taps(strip, w_ref, cin, cout)
        y = acc * a_ref[...][:, None] + b_ref[...][:, None]
        if relu:
            y = jnp.maximum(y, 0.0)
        out_ref[...] = y.reshape(cout, ROWS, SIZE).astype(out_dtype)

    def run(x, w, a, b):
        xspec = lambda f: pl.BlockSpec((cin, ROWS, SIZE), f)
        return pl.pallas_call(
            body,
            grid=(NSTRIP,),
            in_specs=[
                xspec(lambda i: (0, jnp.maximum(i - 1, 0), 0)),
                xspec(lambda i: (0, i, 0)),
                xspec(lambda i: (0, jnp.minimum(i + 1, NSTRIP - 1), 0)),
                pl.BlockSpec((3, 3, cout, cin), lambda i: (0, 0, 0, 0)),
                pl.BlockSpec((cout,), lambda i: (0,)),
                pl.BlockSpec((cout,), lambda i: (0,)),
            ],
            out_specs=pl.BlockSpec((cout, ROWS, SIZE), lambda i: (0, i, 0)),
            out_shape=jax.ShapeDtypeStruct((cout, SIZE, SIZE), out_dtype),
        )(x, x, x, w, a, b)

    return run


def _conv1_body(prev_ref, cur_ref, nxt_ref, w_ref, a_ref, b_ref, out_ref):
    i = pl.program_id(0)
    prev = prev_ref[...].reshape(ROWS, SIZE, EMB).astype(jnp.bfloat16)
    cur = cur_ref[...].reshape(ROWS, SIZE, EMB).astype(jnp.bfloat16)
    nxt = nxt_ref[...].reshape(ROWS, SIZE, EMB).astype(jnp.bfloat16)
    top = jnp.where(i == 0, jnp.zeros((1, SIZE, EMB), jnp.bfloat16),
                    prev[ROWS - 1:ROWS])
    bot = jnp.where(i == NSTRIP - 1, jnp.zeros((1, SIZE, EMB), jnp.bfloat16),
                    nxt[0:1])
    rows = jnp.concatenate([top, cur, bot], axis=0)
    zc = jnp.zeros((ROWS + 2, 1, EMB), jnp.bfloat16)
    strip = jnp.concatenate([zc, rows, zc], axis=1)
    acc = jnp.zeros((512, SPAT), jnp.float32)
    for kh in range(3):
        for kw in range(3):
            xt = strip[kh:kh + ROWS, kw:kw + SIZE, :].reshape(SPAT, EMB)
            acc = acc + lax.dot_general(
                w_ref[kh, kw], xt, (((1,), (1,)), ((), ())),
                preferred_element_type=jnp.float32)
    y = acc * a_ref[...][:, None] + b_ref[...][:, None]
    y = jnp.maximum(y, 0.0)
    out_ref[...] = y.reshape(512, ROWS, SIZE).astype(jnp.bfloat16)


def _conv1(grid, w, a, b):
    gspec = lambda f: pl.BlockSpec((SPAT, EMB), f)
    return pl.pallas_call(
        _conv1_body,
        grid=(NSTRIP,),
        in_specs=[
            gspec(lambda i: (jnp.maximum(i - 1, 0), 0)),
            gspec(lambda i: (i, 0)),
            gspec(lambda i: (jnp.minimum(i + 1, NSTRIP - 1), 0)),
            pl.BlockSpec((3, 3, 512, EMB), lambda i: (0, 0, 0, 0)),
            pl.BlockSpec((512,), lambda i: (0,)),
            pl.BlockSpec((512,), lambda i: (0,)),
        ],
        out_specs=pl.BlockSpec((512, ROWS, SIZE), lambda i: (0, i, 0)),
        out_shape=jax.ShapeDtypeStruct((512, SIZE, SIZE), jnp.bfloat16),
    )(grid, grid, grid, w, a, b)


def _conv4_body(prev_ref, cur_ref, nxt_ref, w_ref, a_ref, b_ref, feat_ref,
                out_ref, y_ref, psum_ref):
    i = pl.program_id(0)
    strip = _strip_chw(prev_ref[...], cur_ref[...], nxt_ref[...], i,
                       256, jnp.bfloat16)
    acc = _conv_taps(strip, w_ref, 256, 256)
    o = acc * a_ref[...][:, None] + b_ref[...][:, None]
    o = jnp.maximum(o, 0.0)
    o3 = o.reshape(256, ROWS, SIZE)
    out_ref[...] = o3
    y_ref[...] = o3 + feat_ref[...]

    @pl.when(i == 0)
    def _():
        psum_ref[...] = jnp.zeros((1, 1), jnp.float32)

    psum_ref[...] += jnp.sum(o).reshape(1, 1)


def _conv4(x, w, a, b, feat):
    xspec = lambda f: pl.BlockSpec((256, ROWS, SIZE), f)
    return pl.pallas_call(
        _conv4_body,
        grid=(NSTRIP,),
        in_specs=[
            xspec(lambda i: (0, jnp.maximum(i - 1, 0), 0)),
            xspec(lambda i: (0, i, 0)),
            xspec(lambda i: (0, jnp.minimum(i + 1, NSTRIP - 1), 0)),
            pl.BlockSpec((3, 3, 256, 256), lambda i: (0, 0, 0, 0)),
            pl.BlockSpec((256,), lambda i: (0,)),
            pl.BlockSpec((256,), lambda i: (0,)),
            xspec(lambda i: (0, i, 0)),
        ],
        out_specs=[
            pl.BlockSpec((256, ROWS, SIZE), lambda i: (0, i, 0)),
            pl.BlockSpec((256, ROWS, SIZE), lambda i: (0, i, 0)),
            pl.BlockSpec((1, 1), lambda i: (0, 0)),
        ],
        out_shape=[
            jax.ShapeDtypeStruct((256, SIZE, SIZE), jnp.float32),
            jax.ShapeDtypeStruct((256, SIZE, SIZE), jnp.float32),
            jax.ShapeDtypeStruct((1, 1), jnp.float32),
        ],
    )(x, x, x, w, a, b, feat)


def _ow_body_first(prev_ref, cur_ref, nxt_ref, w_ref, b_ref, out_ref):
    i = pl.program_id(0)
    strip = _strip_chw(prev_ref[...].astype(jnp.bfloat16),
                       cur_ref[...].astype(jnp.bfloat16),
                       nxt_ref[...].astype(jnp.bfloat16), i, 256, jnp.bfloat16)
    acc = _conv_taps(strip, w_ref, 256, 256)
    y = jnp.maximum(acc + b_ref[...][:, None], 0.0)
    out_ref[...] = y.reshape(256, ROWS, SIZE).astype(jnp.bfloat16)


def _ow_first(x, w, b):
    xspec = lambda f: pl.BlockSpec((256, ROWS, SIZE), f)
    return pl.pallas_call(
        _ow_body_first,
        grid=(NSTRIP,),
        in_specs=[
            xspec(lambda i: (0, jnp.maximum(i - 1, 0), 0)),
            xspec(lambda i: (0, i, 0)),
            xspec(lambda i: (0, jnp.minimum(i + 1, NSTRIP - 1), 0)),
            pl.BlockSpec((3, 3, 256, 256), lambda i: (0, 0, 0, 0)),
            pl.BlockSpec((256,), lambda i: (0,)),
        ],
        out_specs=pl.BlockSpec((256, ROWS, SIZE), lambda i: (0, i, 0)),
        out_shape=jax.ShapeDtypeStruct((256, SIZE, SIZE), jnp.bfloat16),
    )(x, x, x, w, b)


def _ow4_body(x_ref, w_ref, b_ref, out_ref):
    xt = x_ref[...].reshape(256, SPAT)
    acc = lax.dot_general(w_ref[...], xt, (((1,), (0,)), ((), ())),
                          preferred_element_type=jnp.float32)
    acc = acc + b_ref[...][:, None]
    out_ref[...] = acc.reshape(256, ROWS, SIZE)


def _ow4(x, w, b):
    return pl.pallas_call(
        _ow4_body,
        grid=(NSTRIP,),
        in_specs=[
            pl.BlockSpec((256, ROWS, SIZE), lambda i: (0, i, 0)),
            pl.BlockSpec((256, 256), lambda i: (0, 0)),
            pl.BlockSpec((256,), lambda i: (0,)),
        ],
        out_specs=pl.BlockSpec((256, ROWS, SIZE), lambda i: (0, i, 0)),
        out_shape=jax.ShapeDtypeStruct((256, SIZE, SIZE), jnp.float32),
    )(x, w, b)


def _prep_w(w):
    return jnp.transpose(w, (2, 3, 0, 1)).astype(jnp.bfloat16)


def kernel(features, target_points, visible_points, transform, all_pts_mask,
           params):
    p = params
    inv_s = 1.0 / jnp.sqrt(jnp.float32(1.0 + 1e-5))
    tp = target_points[0]
    tp8 = jnp.concatenate(
        [tp, jnp.ones((NV, 1), jnp.float32),
         jnp.zeros((NV, EMB - 4), jnp.float32)], axis=1)
    vis = visible_points[0].astype(jnp.int32)
    apm = all_pts_mask[0].astype(jnp.int32)
    filler = p['filler'].astype(jnp.float32)

    tof0, xyz8 = _sc_gather(p['emb'].astype(jnp.float32), tp8, vis)
    tofp, pcoord, pidx0, pidx1, aidx0, aidx1 = _geom(
        xyz8, transform[0], tof0, filler, apm)
    interp = _knn(apm.astype(jnp.float32), pcoord, tofp)

    fillrow = filler.reshape(1, EMB)
    fillblk = jnp.broadcast_to(filler[None, :], (128, EMB))
    grid = _sc_scatter(interp, tofp, fillrow, fillblk,
                       aidx0, aidx1, pidx0, pidx1)

    def ab(g, cb, be):
        a = g * inv_s
        return a, cb * a + be

    a1, b1 = ab(p['g1'], p['cb1'], p['be1'])
    a2, b2 = ab(p['g2'], p['cb2'], p['be2'])
    a3, b3 = ab(p['g3'], p['cb3'], p['be3'])
    a4, b4 = ab(p['g4'], p['cb4'], p['be4'])

    x1 = _conv1(grid, _prep_w(p['cw1']), a1, b1)
    x2 = _make_conv_chw(512, 512, True, jnp.bfloat16)(
        x1, _prep_w(p['cw2']), a2, b2)
    x3 = _make_conv_chw(512, 256, True, jnp.bfloat16)(
        x2, _prep_w(p['cw3']), a3, b3)
    feat = features[0]
    output, y, psum = _conv4(x3, _prep_w(p['cw4']), a4, b4, feat)

    z1 = _ow_first(y, _prep_w(p['ow1']), p['ob1'])
    z2 = _make_conv_chw(256, 256, True, jnp.bfloat16)(
        z1, _prep_w(p['ow2']), jnp.ones((256,), jnp.float32), p['ob2'])
    z3 = _make_conv_chw(256, 256, True, jnp.bfloat16)(
        z2, _prep_w(p['ow3']), jnp.ones((256,), jnp.float32), p['ob3'])
    processed = _ow4(z3, p['ow4'][:, :, 0, 0].astype(jnp.bfloat16), p['ob4'])

    return (processed[None], output[None], y[None], psum[0, 0])


# speedup vs baseline: 1.8297x; 1.8297x over previous
import functools

import jax
import jax.numpy as jnp
from jax import lax
from jax.experimental import pallas as pl
from jax.experimental.pallas import tpu as pltpu
from jax.experimental.pallas import tpu_sc as plsc

EMB = 128
NV = 10000
KNN = 11
SIZE = 296
NPIX = SIZE * SIZE
CHUNK = 2744
GRID_ROWS = 32 * CHUNK
HALF = 16 * CHUNK
TRASH0 = NPIX
TRASH1 = NPIX + 8
NQ = 4096
NS = 2048
QBLK = 512
ROWS = 8
NSTRIP = SIZE // ROWS
SPAT = ROWS * SIZE


def _sc_gather_body(emb_hbm, tp8_hbm, vis_hbm, tof_hbm, xyz_hbm,
                    idx_v, emb_v, xyz_v, sem):
    c = lax.axis_index("c")
    s = lax.axis_index("s")
    wid = c * 16 + s
    base = wid * 64
    pltpu.sync_copy(vis_hbm.at[pl.ds(base, 64)], idx_v)
    pltpu.async_copy(emb_hbm.at[idx_v], emb_v, sem).wait()
    pltpu.sync_copy(emb_v, tof_hbm.at[pl.ds(base, 64)])
    pltpu.async_copy(tp8_hbm.at[idx_v], xyz_v, sem).wait()
    pltpu.sync_copy(xyz_v, xyz_hbm.at[pl.ds(base, 64)])


def _sc_gather(emb, tp8, vis):
    mesh = plsc.VectorSubcoreMesh(core_axis_name="c", subcore_axis_name="s")
    kern = functools.partial(
        pl.kernel,
        mesh=mesh,
        compiler_params=pltpu.CompilerParams(use_tc_tiling_on_sc=True),
        out_type=[
            jax.ShapeDtypeStruct((NS, EMB), jnp.float32),
            jax.ShapeDtypeStruct((NS, EMB), jnp.float32),
        ],
        scratch_types=[
            pltpu.VMEM((64,), jnp.int32),
            pltpu.VMEM((64, EMB), jnp.float32),
            pltpu.VMEM((64, EMB), jnp.float32),
            pltpu.SemaphoreType.DMA,
        ],
    )(_sc_gather_body)
    return kern(emb, tp8, vis)


def _sc_scatter_body(interp_hbm, tofp_hbm, fillrow_hbm, fillblk_hbm,
                     aidx0_hbm, aidx1_hbm, pidx0_hbm, pidx1_hbm, grid_hbm,
                     fill_v, idxa_v, rowsa_v, idxp_v, rowsp_v, sem):
    c = lax.axis_index("c")
    s = lax.axis_index("s")
    pltpu.sync_copy(fillblk_hbm, fill_v)
    fbase = (c * 16 + s) * CHUNK
    copies = []
    for j in range(CHUNK // 128):
        copies.append(pltpu.async_copy(
            fill_v, grid_hbm.at[pl.ds(fbase + j * 128, 128)], sem))
    rem = CHUNK - 128 * (CHUNK // 128)
    copies.append(pltpu.async_copy(
        fill_v.at[pl.ds(0, rem)],
        grid_hbm.at[pl.ds(fbase + 128 * (CHUNK // 128), rem)], sem))
    for cp in copies:
        cp.wait()
    plsc.subcore_barrier()
    abase = s * 256

    @pl.when(c == 0)
    def _():
        for j in range(2):
            pltpu.sync_copy(aidx0_hbm.at[pl.ds(abase + j * 128, 128)],
                            idxa_v.at[j])

    @pl.when(c == 1)
    def _():
        for j in range(2):
            pltpu.sync_copy(aidx1_hbm.at[pl.ds(abase + j * 128, 128)],
                            idxa_v.at[j])

    for j in range(2):
        pltpu.sync_copy(interp_hbm.at[pl.ds(abase + j * 128, 128)],
                        rowsa_v.at[j])
        pltpu.async_copy(rowsa_v.at[j], grid_hbm.at[idxa_v.at[j]], sem).wait()
    plsc.subcore_barrier()
    pbase = s * 128

    @pl.when(c == 0)
    def _():
        pltpu.sync_copy(pidx0_hbm.at[pl.ds(pbase, 128)], idxp_v)

    @pl.when(c == 1)
    def _():
        pltpu.sync_copy(pidx1_hbm.at[pl.ds(pbase, 128)], idxp_v)

    pltpu.sync_copy(tofp_hbm.at[pl.ds(pbase, 128)], rowsp_v)
    pltpu.async_copy(rowsp_v, grid_hbm.at[idxp_v], sem).wait()
    plsc.subcore_barrier()
    @pl.when((c == 0) & (s == 0))
    def _():
        pltpu.sync_copy(fillrow_hbm, grid_hbm.at[pl.ds(0, 1)])


def _sc_scatter(interp, tofp, fillrow, fillblk, aidx0, aidx1, pidx0, pidx1):
    mesh = plsc.VectorSubcoreMesh(core_axis_name="c", subcore_axis_name="s")
    kern = functools.partial(
        pl.kernel,
        mesh=mesh,
        compiler_params=pltpu.CompilerParams(use_tc_tiling_on_sc=True),
        out_type=jax.ShapeDtypeStruct((GRID_ROWS, EMB), jnp.float32),
        scratch_types=[
            pltpu.VMEM((128, EMB), jnp.float32),
            pltpu.VMEM((2, 128), jnp.int32),
            pltpu.VMEM((2, 128, EMB), jnp.float32),
            pltpu.VMEM((128,), jnp.int32),
            pltpu.VMEM((128, EMB), jnp.float32),
            pltpu.SemaphoreType.DMA,
        ],
    )(_sc_scatter_body)
    return kern(interp, tofp, fillrow, fillblk, aidx0, aidx1, pidx0, pidx1)


def _geom_body(xyz_ref, tr_ref, tof0_ref, fill_ref, apm_ref,
               tofp_ref, pcoord_ref, pidx0_ref, pidx1_ref,
               aidx0_ref, aidx1_ref):
    ph = xyz_ref[:, 0:4]
    tr = tr_ref[...]
    cam = lax.dot_general(ph, tr, (((1,), (1,)), ((), ())),
                          preferred_element_type=jnp.float32)
    x = cam[:, 0]
    y = cam[:, 1]
    z = cam[:, 2]
    zs = jnp.where(jnp.abs(z) < 1e-6, 1e-6, z)
    u = 12.0 * x / zs + SIZE / 2.0
    v = 12.0 * y / zs + SIZE / 2.0
    px = jnp.round(u)
    py = jnp.round(v)
    bad = (px < 0.0) | (px > 295.0) | (py < 0.0) | (py > 295.0)
    px = jnp.where(bad, 0.0, px)
    py = jnp.where(bad, 0.0, py)
    tofp_ref[...] = jnp.where(bad[:, None], fill_ref[...][None, :],
                              tof0_ref[...])
    pcoord_ref[0:1, :] = px.reshape(1, NS)
    pcoord_ref[1:2, :] = py.reshape(1, NS)
    pcoord_ref[2:8, :] = jnp.zeros((6, NS), jnp.float32)
    pid = py.astype(jnp.int32) * SIZE + px.astype(jnp.int32)
    pid_row = pid.reshape(1, NS)
    col = lax.broadcasted_iota(jnp.int32, (1, NS), 1)
    loser_parts = []
    for cblk in range(4):
        pc_chunk = pid[cblk * 512:(cblk + 1) * 512]
        row_ids = lax.broadcasted_iota(jnp.int32, (512, 1), 0) + cblk * 512
        eq = (pc_chunk[:, None] == pid_row) & (col > row_ids)
        loser_parts.append(jnp.any(eq, axis=1).astype(jnp.int32))
    loser = jnp.concatenate(loser_parts, axis=0)
    own0 = pid < HALF
    keep = loser == 0
    pidx0_ref[...] = jnp.where(own0 & keep, pid, TRASH0)
    pidx1_ref[...] = jnp.where(jnp.logical_not(own0) & keep, pid, TRASH1)
    apm = apm_ref[...]
    aid = apm[:, 1] * SIZE + apm[:, 0]
    aown0 = aid < HALF
    aidx0_ref[...] = jnp.where(aown0, aid, TRASH0)
    aidx1_ref[...] = jnp.where(jnp.logical_not(aown0), aid, TRASH1)


def _geom(xyz8, transform, tof0, filler, apm):
    return pl.pallas_call(
        _geom_body,
        out_shape=[
            jax.ShapeDtypeStruct((NS, EMB), jnp.float32),
            jax.ShapeDtypeStruct((8, NS), jnp.float32),
            jax.ShapeDtypeStruct((NS,), jnp.int32),
            jax.ShapeDtypeStruct((NS,), jnp.int32),
            jax.ShapeDtypeStruct((NQ,), jnp.int32),
            jax.ShapeDtypeStruct((NQ,), jnp.int32),
        ],
    )(xyz8, transform, tof0, filler, apm)


def _knn_body(apmf_ref, pcoord_ref, tofp_ref, out_ref):
    qx = apmf_ref[:, 0:1]
    qy = apmf_ref[:, 1:2]
    px = pcoord_ref[0:1, :]
    py = pcoord_ref[1:2, :]
    dx = qx - px
    dy = qy - py
    d2 = dx * dx + dy * dy
    col = lax.broadcasted_iota(jnp.int32, (QBLK, NS), 1)
    key = d2.astype(jnp.int32) * NS + col
    wmat = jnp.zeros((QBLK, NS), jnp.float32)
    wsum = jnp.zeros((QBLK, 1), jnp.float32)
    big = jnp.int32(2147483647)
    for _ in range(KNN):
        m = jnp.min(key, axis=1, keepdims=True)
        sel = key == m
        d2f = (m // NS).astype(jnp.float32)
        wt = 1.0 / (jnp.sqrt(d2f + 1e-12) + 1e-8)
        wmat = wmat + jnp.where(sel, wt, 0.0)
        wsum = wsum + wt
        key = jnp.where(sel, big, key)
    wmat = wmat / wsum
    out_ref[...] = jnp.dot(wmat, tofp_ref[...],
                           preferred_element_type=jnp.float32)


def _knn(apmf, pcoord, tofp):
    return pl.pallas_call(
        _knn_body,
        grid=(NQ // QBLK,),
        in_specs=[
            pl.BlockSpec((QBLK, 2), lambda i: (i, 0)),
            pl.BlockSpec((8, NS), lambda i: (0, 0)),
            pl.BlockSpec((NS, EMB), lambda i: (0, 0)),
        ],
        out_specs=pl.BlockSpec((QBLK, EMB), lambda i: (i, 0)),
        out_shape=jax.ShapeDtypeStruct((NQ, EMB), jnp.float32),
    )(apmf, pcoord, tofp)


def _strip_hwc(prev, cur, nxt, i, cin):
    top = jnp.where(i == 0, jnp.zeros((1, SIZE, cin), jnp.bfloat16),
                    prev[ROWS - 1:ROWS])
    bot = jnp.where(i == NSTRIP - 1, jnp.zeros((1, SIZE, cin), jnp.bfloat16),
                    nxt[0:1])
    rows = jnp.concatenate([top, cur, bot], axis=0)
    zc = jnp.zeros((ROWS + 2, 1, cin), jnp.bfloat16)
    return jnp.concatenate([zc, rows, zc], axis=1)


def _conv_taps_hwc(strip, w_ref, cin, cout):
    acc = jnp.zeros((SPAT, cout), jnp.float32)
    for kw in range(3):
        xs = strip[:, kw:kw + SIZE, :]
        for kh in range(3):
            xt = xs[kh:kh + ROWS].reshape(SPAT, cin)
            acc = acc + lax.dot_general(
                xt, w_ref[kh, kw], (((1,), (0,)), ((), ())),
                preferred_element_type=jnp.float32)
    return acc


def _make_conv_hwc(cin, cout, relu, out_dtype, in_f32=False):
    def body(prev_ref, cur_ref, nxt_ref, w_ref, a_ref, b_ref, out_ref):
        i = pl.program_id(0)
        if in_f32:
            prev = prev_ref[...].astype(jnp.bfloat16)
            cur = cur_ref[...].astype(jnp.bfloat16)
            nxt = nxt_ref[...].astype(jnp.bfloat16)
        else:
            prev, cur, nxt = prev_ref[...], cur_ref[...], nxt_ref[...]
        strip = _strip_hwc(prev, cur, nxt, i, cin)
        acc = _conv_taps_hwc(strip, w_ref, cin, cout)
        y = acc * a_ref[...][None, :] + b_ref[...][None, :]
        if relu:
            y = jnp.maximum(y, 0.0)
        out_ref[...] = y.reshape(ROWS, SIZE, cout).astype(out_dtype)

    def run(x, w, a, b):
        xspec = lambda f: pl.BlockSpec((ROWS, SIZE, cin), f)
        return pl.pallas_call(
            body,
            grid=(NSTRIP,),
            in_specs=[
                xspec(lambda i: (jnp.maximum(i - 1, 0), 0, 0)),
                xspec(lambda i: (i, 0, 0)),
                xspec(lambda i: (jnp.minimum(i + 1, NSTRIP - 1), 0, 0)),
                pl.BlockSpec((3, 3, cin, cout), lambda i: (0, 0, 0, 0)),
                pl.BlockSpec((cout,), lambda i: (0,)),
                pl.BlockSpec((cout,), lambda i: (0,)),
            ],
            out_specs=pl.BlockSpec((ROWS, SIZE, cout), lambda i: (i, 0, 0)),
            out_shape=jax.ShapeDtypeStruct((SIZE, SIZE, cout), out_dtype),
        )(x, x, x, w, a, b)

    return run


def _conv1_body(prev_ref, cur_ref, nxt_ref, w_ref, a_ref, b_ref, out_ref):
    i = pl.program_id(0)
    prev = prev_ref[...].astype(jnp.bfloat16)
    cur = cur_ref[...].reshape(ROWS, SIZE, EMB).astype(jnp.bfloat16)
    nxt = nxt_ref[...].astype(jnp.bfloat16)
    top = jnp.where(i == 0, jnp.zeros((1, SIZE, EMB), jnp.bfloat16),
                    prev.reshape(1, SIZE, EMB))
    bot = jnp.where(i == NSTRIP - 1, jnp.zeros((1, SIZE, EMB), jnp.bfloat16),
                    nxt.reshape(1, SIZE, EMB))
    rows = jnp.concatenate([top, cur, bot], axis=0)
    zc = jnp.zeros((ROWS + 2, 1, EMB), jnp.bfloat16)
    strip = jnp.concatenate([zc, rows, zc], axis=1)
    acc = _conv_taps_hwc(strip, w_ref, EMB, 512)
    y = acc * a_ref[...][None, :] + b_ref[...][None, :]
    y = jnp.maximum(y, 0.0)
    out_ref[...] = y.reshape(ROWS, SIZE, 512).astype(jnp.bfloat16)


def _conv1(grid, w, a, b):
    return pl.pallas_call(
        _conv1_body,
        grid=(NSTRIP,),
        in_specs=[
            pl.BlockSpec((SIZE, EMB),
                         lambda i: (jnp.maximum(i - 1, 0) * ROWS + ROWS - 1, 0)),
            pl.BlockSpec((SPAT, EMB), lambda i: (i, 0)),
            pl.BlockSpec((SIZE, EMB),
                         lambda i: (jnp.minimum(i + 1, NSTRIP - 1) * ROWS, 0)),
            pl.BlockSpec((3, 3, EMB, 512), lambda i: (0, 0, 0, 0)),
            pl.BlockSpec((512,), lambda i: (0,)),
            pl.BlockSpec((512,), lambda i: (0,)),
        ],
        out_specs=pl.BlockSpec((ROWS, SIZE, 512), lambda i: (i, 0, 0)),
        out_shape=jax.ShapeDtypeStruct((SIZE, SIZE, 512), jnp.bfloat16),
    )(grid, grid, grid, w, a, b)


def _conv4_body(prev_ref, cur_ref, nxt_ref, w_ref, a_ref, b_ref, feat_ref,
                out_ref, y_ref, psum_ref):
    i = pl.program_id(0)
    strip = _strip_hwc(prev_ref[...], cur_ref[...], nxt_ref[...], i, 256)
    acc = _conv_taps_hwc(strip, w_ref, 256, 256)
    o = acc * a_ref[...][None, :] + b_ref[...][None, :]
    o = jnp.maximum(o, 0.0)
    o3 = o.reshape(ROWS, SIZE, 256)
    out_ref[...] = o3
    y_ref[...] = o3 + feat_ref[...]

    @pl.when(i == 0)
    def _():
        psum_ref[...] = jnp.zeros((1, 1), jnp.float32)

    psum_ref[...] += jnp.sum(o).reshape(1, 1)


def _conv4(x, w, a, b, feat):
    xspec = lambda f: pl.BlockSpec((ROWS, SIZE, 256), f)
    return pl.pallas_call(
        _conv4_body,
        grid=(NSTRIP,),
        in_specs=[
            xspec(lambda i: (jnp.maximum(i - 1, 0), 0, 0)),
            xspec(lambda i: (i, 0, 0)),
            xspec(lambda i: (jnp.minimum(i + 1, NSTRIP - 1), 0, 0)),
            pl.BlockSpec((3, 3, 256, 256), lambda i: (0, 0, 0, 0)),
            pl.BlockSpec((256,), lambda i: (0,)),
            pl.BlockSpec((256,), lambda i: (0,)),
            xspec(lambda i: (i, 0, 0)),
        ],
        out_specs=[
            pl.BlockSpec((ROWS, SIZE, 256), lambda i: (i, 0, 0)),
            pl.BlockSpec((ROWS, SIZE, 256), lambda i: (i, 0, 0)),
            pl.BlockSpec((1, 1), lambda i: (0, 0)),
        ],
        out_shape=[
            jax.ShapeDtypeStruct((SIZE, SIZE, 256), jnp.float32),
            jax.ShapeDtypeStruct((SIZE, SIZE, 256), jnp.float32),
            jax.ShapeDtypeStruct((1, 1), jnp.float32),
        ],
    )(x, x, x, w, a, b, feat)


def _ow4_body(x_ref, w_ref, b_ref, out_ref):
    xt = x_ref[...].reshape(SPAT, 256)
    acc = lax.dot_general(xt, w_ref[...], (((1,), (0,)), ((), ())),
                          preferred_element_type=jnp.float32)
    acc = acc + b_ref[...][None, :]
    out_ref[...] = acc.reshape(ROWS, SIZE, 256)


def _ow4(x, w, b):
    return pl.pallas_call(
        _ow4_body,
        grid=(NSTRIP,),
        in_specs=[
            pl.BlockSpec((ROWS, SIZE, 256), lambda i: (i, 0, 0)),
            pl.BlockSpec((256, 256), lambda i: (0, 0)),
            pl.BlockSpec((256,), lambda i: (0,)),
        ],
        out_specs=pl.BlockSpec((ROWS, SIZE, 256), lambda i: (i, 0, 0)),
        out_shape=jax.ShapeDtypeStruct((SIZE, SIZE, 256), jnp.float32),
    )(x, w, b)


def _prep_w(w):
    return jnp.transpose(w, (2, 3, 1, 0)).astype(jnp.bfloat16)


def kernel(features, target_points, visible_points, transform, all_pts_mask,
           params):
    p = params
    inv_s = 1.0 / jnp.sqrt(jnp.float32(1.0 + 1e-5))
    tp = target_points[0]
    tp8 = jnp.concatenate(
        [tp, jnp.ones((NV, 1), jnp.float32),
         jnp.zeros((NV, EMB - 4), jnp.float32)], axis=1)
    vis = visible_points[0].astype(jnp.int32)
    apm = all_pts_mask[0].astype(jnp.int32)
    filler = p['filler'].astype(jnp.float32)

    tof0, xyz8 = _sc_gather(p['emb'].astype(jnp.float32), tp8, vis)
    tofp, pcoord, pidx0, pidx1, aidx0, aidx1 = _geom(
        xyz8, transform[0], tof0, filler, apm)
    interp = _knn(apm.astype(jnp.float32), pcoord, tofp)

    fillrow = filler.reshape(1, EMB)
    fillblk = jnp.broadcast_to(filler[None, :], (128, EMB))
    grid = _sc_scatter(interp, tofp, fillrow, fillblk,
                       aidx0, aidx1, pidx0, pidx1)

    def ab(g, cb, be):
        a = g * inv_s
        return a, cb * a + be

    a1, b1 = ab(p['g1'], p['cb1'], p['be1'])
    a2, b2 = ab(p['g2'], p['cb2'], p['be2'])
    a3, b3 = ab(p['g3'], p['cb3'], p['be3'])
    a4, b4 = ab(p['g4'], p['cb4'], p['be4'])

    x1 = _conv1(grid, _prep_w(p['cw1']), a1, b1)
    x2 = _make_conv_hwc(512, 512, True, jnp.bfloat16)(
        x1, _prep_w(p['cw2']), a2, b2)
    x3 = _make_conv_hwc(512, 256, True, jnp.bfloat16)(
        x2, _prep_w(p['cw3']), a3, b3)
    feat = jnp.transpose(features[0], (1, 2, 0))
    output, y, psum = _conv4(x3, _prep_w(p['cw4']), a4, b4, feat)

    ones = jnp.ones((256,), jnp.float32)
    z1 = _make_conv_hwc(256, 256, True, jnp.bfloat16, in_f32=True)(
        y, _prep_w(p['ow1']), ones, p['ob1'])
    z2 = _make_conv_hwc(256, 256, True, jnp.bfloat16)(
        z1, _prep_w(p['ow2']), ones, p['ob2'])
    z3 = _make_conv_hwc(256, 256, True, jnp.bfloat16)(
        z2, _prep_w(p['ow3']), ones, p['ob3'])
    processed = _ow4(z3, jnp.transpose(p['ow4'][:, :, 0, 0]).astype(jnp.bfloat16),
                     p['ob4'])

    def tr(o):
        return jnp.transpose(o, (2, 0, 1))[None]

    return (tr(processed), tr(output), tr(y), psum[0, 0])


# speedup vs baseline: 1.8633x; 1.0184x over previous
import functools

import jax
import jax.numpy as jnp
from jax import lax
from jax.experimental import pallas as pl
from jax.experimental.pallas import tpu as pltpu
from jax.experimental.pallas import tpu_sc as plsc

EMB = 128
NV = 10000
KNN = 11
SIZE = 296
NPIX = SIZE * SIZE
CHUNK = 2744
GRID_ROWS = 32 * CHUNK
HALF = 16 * CHUNK
TRASH0 = NPIX
TRASH1 = NPIX + 8
NQ = 4096
NS = 2048
QBLK = 512
ROWS = 8
NSTRIP = SIZE // ROWS
SPAT = ROWS * SIZE


def _sc_gather_body(emb_hbm, tp8_hbm, vis_hbm, tof_hbm, xyz_hbm,
                    idx_v, emb_v, xyz_v, sem):
    c = lax.axis_index("c")
    s = lax.axis_index("s")
    wid = c * 16 + s
    base = wid * 64
    pltpu.sync_copy(vis_hbm.at[pl.ds(base, 64)], idx_v)
    pltpu.async_copy(emb_hbm.at[idx_v], emb_v, sem).wait()
    pltpu.sync_copy(emb_v, tof_hbm.at[pl.ds(base, 64)])
    pltpu.async_copy(tp8_hbm.at[idx_v], xyz_v, sem).wait()
    pltpu.sync_copy(xyz_v, xyz_hbm.at[pl.ds(base, 64)])


def _sc_gather(emb, tp8, vis):
    mesh = plsc.VectorSubcoreMesh(core_axis_name="c", subcore_axis_name="s")
    kern = functools.partial(
        pl.kernel,
        mesh=mesh,
        compiler_params=pltpu.CompilerParams(use_tc_tiling_on_sc=True),
        out_type=[
            jax.ShapeDtypeStruct((NS, EMB), jnp.float32),
            jax.ShapeDtypeStruct((NS, EMB), jnp.float32),
        ],
        scratch_types=[
            pltpu.VMEM((64,), jnp.int32),
            pltpu.VMEM((64, EMB), jnp.float32),
            pltpu.VMEM((64, EMB), jnp.float32),
            pltpu.SemaphoreType.DMA,
        ],
    )(_sc_gather_body)
    return kern(emb, tp8, vis)


def _sc_scatter_body(interp_hbm, tofp_hbm, fillrow_hbm, fillblk_hbm,
                     aidx0_hbm, aidx1_hbm, pidx0_hbm, pidx1_hbm, grid_hbm,
                     fill_v, idxa_v, rowsa_v, idxp_v, rowsp_v, sem):
    c = lax.axis_index("c")
    s = lax.axis_index("s")
    pltpu.sync_copy(fillblk_hbm, fill_v.at[pl.ds(0, 128)])
    pltpu.sync_copy(fillblk_hbm, fill_v.at[pl.ds(128, 128)])
    fbase = (c * 16 + s) * CHUNK
    copies = []
    for j in range(CHUNK // 256):
        copies.append(pltpu.async_copy(
            fill_v, grid_hbm.at[pl.ds(fbase + j * 256, 256)], sem))
    rem = CHUNK - 256 * (CHUNK // 256)
    copies.append(pltpu.async_copy(
        fill_v.at[pl.ds(0, rem)],
        grid_hbm.at[pl.ds(fbase + 256 * (CHUNK // 256), rem)], sem))
    for cp in copies:
        cp.wait()
    plsc.subcore_barrier()
    abase = s * 256

    @pl.when(c == 0)
    def _():
        for j in range(2):
            pltpu.sync_copy(aidx0_hbm.at[pl.ds(abase + j * 128, 128)],
                            idxa_v.at[j])

    @pl.when(c == 1)
    def _():
        for j in range(2):
            pltpu.sync_copy(aidx1_hbm.at[pl.ds(abase + j * 128, 128)],
                            idxa_v.at[j])

    for j in range(2):
        pltpu.sync_copy(interp_hbm.at[pl.ds(abase + j * 128, 128)],
                        rowsa_v.at[j])
        pltpu.async_copy(rowsa_v.at[j], grid_hbm.at[idxa_v.at[j]], sem).wait()
    plsc.subcore_barrier()
    pbase = s * 128

    @pl.when(c == 0)
    def _():
        pltpu.sync_copy(pidx0_hbm.at[pl.ds(pbase, 128)], idxp_v)

    @pl.when(c == 1)
    def _():
        pltpu.sync_copy(pidx1_hbm.at[pl.ds(pbase, 128)], idxp_v)

    pltpu.sync_copy(tofp_hbm.at[pl.ds(pbase, 128)], rowsp_v)
    pltpu.async_copy(rowsp_v, grid_hbm.at[idxp_v], sem).wait()
    plsc.subcore_barrier()
    @pl.when((c == 0) & (s == 0))
    def _():
        pltpu.sync_copy(fillrow_hbm, grid_hbm.at[pl.ds(0, 1)])


def _sc_scatter(interp, tofp, fillrow, fillblk, aidx0, aidx1, pidx0, pidx1):
    mesh = plsc.VectorSubcoreMesh(core_axis_name="c", subcore_axis_name="s")
    kern = functools.partial(
        pl.kernel,
        mesh=mesh,
        compiler_params=pltpu.CompilerParams(use_tc_tiling_on_sc=True),
        out_type=jax.ShapeDtypeStruct((GRID_ROWS, EMB), jnp.float32),
        scratch_types=[
            pltpu.VMEM((256, EMB), jnp.float32),
            pltpu.VMEM((2, 128), jnp.int32),
            pltpu.VMEM((2, 128, EMB), jnp.float32),
            pltpu.VMEM((128,), jnp.int32),
            pltpu.VMEM((128, EMB), jnp.float32),
            pltpu.SemaphoreType.DMA,
        ],
    )(_sc_scatter_body)
    return kern(interp, tofp, fillrow, fillblk, aidx0, aidx1, pidx0, pidx1)


def _geom_body(xyz_ref, tr_ref, tof0_ref, fill_ref, apm_ref,
               tofp_ref, pcoord_ref, pidx0_ref, pidx1_ref,
               aidx0_ref, aidx1_ref):
    ph = xyz_ref[:, 0:4]
    tr = tr_ref[...]
    cam = lax.dot_general(ph, tr, (((1,), (1,)), ((), ())),
                          preferred_element_type=jnp.float32)
    x = cam[:, 0]
    y = cam[:, 1]
    z = cam[:, 2]
    zs = jnp.where(jnp.abs(z) < 1e-6, 1e-6, z)
    u = 12.0 * x / zs + SIZE / 2.0
    v = 12.0 * y / zs + SIZE / 2.0
    px = jnp.round(u)
    py = jnp.round(v)
    bad = (px < 0.0) | (px > 295.0) | (py < 0.0) | (py > 295.0)
    px = jnp.where(bad, 0.0, px)
    py = jnp.where(bad, 0.0, py)
    tofp_ref[...] = jnp.where(bad[:, None], fill_ref[...][None, :],
                              tof0_ref[...])
    pcoord_ref[0:1, :] = px.reshape(1, NS)
    pcoord_ref[1:2, :] = py.reshape(1, NS)
    pcoord_ref[2:8, :] = jnp.zeros((6, NS), jnp.float32)
    pid = py.astype(jnp.int32) * SIZE + px.astype(jnp.int32)
    pid_row = pid.reshape(1, NS)
    col = lax.broadcasted_iota(jnp.int32, (1, NS), 1)
    loser_parts = []
    for cblk in range(4):
        pc_chunk = pid[cblk * 512:(cblk + 1) * 512]
        row_ids = lax.broadcasted_iota(jnp.int32, (512, 1), 0) + cblk * 512
        eq = (pc_chunk[:, None] == pid_row) & (col > row_ids)
        loser_parts.append(jnp.any(eq, axis=1).astype(jnp.int32))
    loser = jnp.concatenate(loser_parts, axis=0)
    own0 = pid < HALF
    keep = loser == 0
    pidx0_ref[...] = jnp.where(own0 & keep, pid, TRASH0)
    pidx1_ref[...] = jnp.where(jnp.logical_not(own0) & keep, pid, TRASH1)
    apm = apm_ref[...]
    aid = apm[:, 1] * SIZE + apm[:, 0]
    aown0 = aid < HALF
    aidx0_ref[...] = jnp.where(aown0, aid, TRASH0)
    aidx1_ref[...] = jnp.where(jnp.logical_not(aown0), aid, TRASH1)


def _geom(xyz8, transform, tof0, filler, apm):
    return pl.pallas_call(
        _geom_body,
        out_shape=[
            jax.ShapeDtypeStruct((NS, EMB), jnp.float32),
            jax.ShapeDtypeStruct((8, NS), jnp.float32),
            jax.ShapeDtypeStruct((NS,), jnp.int32),
            jax.ShapeDtypeStruct((NS,), jnp.int32),
            jax.ShapeDtypeStruct((NQ,), jnp.int32),
            jax.ShapeDtypeStruct((NQ,), jnp.int32),
        ],
    )(xyz8, transform, tof0, filler, apm)


def _knn_body(apmf_ref, pcoord_ref, tofp_ref, out_ref):
    qx = apmf_ref[:, 0:1]
    qy = apmf_ref[:, 1:2]
    px = pcoord_ref[0:1, :]
    py = pcoord_ref[1:2, :]
    dx = qx - px
    dy = qy - py
    d2 = dx * dx + dy * dy
    col = lax.broadcasted_iota(jnp.int32, (QBLK, NS), 1)
    key = d2.astype(jnp.int32) * NS + col
    big = jnp.int32(2147483647)
    kw = key
    for _ in range(KNN - 1):
        m = jnp.min(kw, axis=1, keepdims=True)
        kw = jnp.where(kw == m, big, kw)
    t11 = jnp.min(kw, axis=1, keepdims=True)
    wt = 1.0 / (jnp.sqrt(d2 + 1e-12) + 1e-8)
    wmat = jnp.where(key <= t11, wt, 0.0)
    wsum = jnp.sum(wmat, axis=1, keepdims=True)
    wmat = wmat / wsum
    out_ref[...] = jnp.dot(wmat, tofp_ref[...],
                           preferred_element_type=jnp.float32)


def _knn(apmf, pcoord, tofp):
    return pl.pallas_call(
        _knn_body,
        grid=(NQ // QBLK,),
        in_specs=[
            pl.BlockSpec((QBLK, 2), lambda i: (i, 0)),
            pl.BlockSpec((8, NS), lambda i: (0, 0)),
            pl.BlockSpec((NS, EMB), lambda i: (0, 0)),
        ],
        out_specs=pl.BlockSpec((QBLK, EMB), lambda i: (i, 0)),
        out_shape=jax.ShapeDtypeStruct((NQ, EMB), jnp.float32),
    )(apmf, pcoord, tofp)


def _strip_hwc(prev, cur, nxt, i, cin):
    top = jnp.where(i == 0, jnp.zeros((1, SIZE, cin), jnp.bfloat16),
                    prev[ROWS - 1:ROWS])
    bot = jnp.where(i == NSTRIP - 1, jnp.zeros((1, SIZE, cin), jnp.bfloat16),
                    nxt[0:1])
    rows = jnp.concatenate([top, cur, bot], axis=0)
    zc = jnp.zeros((ROWS + 2, 1, cin), jnp.bfloat16)
    return jnp.concatenate([zc, rows, zc], axis=1)


def _conv_taps_hwc(strip, w_ref, cin, cout):
    acc = jnp.zeros((SPAT, cout), jnp.float32)
    for kw in range(3):
        xs = strip[:, kw:kw + SIZE, :]
        for kh in range(3):
            xt = xs[kh:kh + ROWS].reshape(SPAT, cin)
            acc = acc + lax.dot_general(
                xt, w_ref[kh, kw], (((1,), (0,)), ((), ())),
                preferred_element_type=jnp.float32)
    return acc


def _make_conv_hwc(cin, cout, relu, out_dtype, in_f32=False):
    def body(prev_ref, cur_ref, nxt_ref, w_ref, a_ref, b_ref, out_ref):
        i = pl.program_id(0)
        if in_f32:
            prev = prev_ref[...].astype(jnp.bfloat16)
            cur = cur_ref[...].astype(jnp.bfloat16)
            nxt = nxt_ref[...].astype(jnp.bfloat16)
        else:
            prev, cur, nxt = prev_ref[...], cur_ref[...], nxt_ref[...]
        strip = _strip_hwc(prev, cur, nxt, i, cin)
        acc = _conv_taps_hwc(strip, w_ref, cin, cout)
        y = acc * a_ref[...][None, :] + b_ref[...][None, :]
        if relu:
            y = jnp.maximum(y, 0.0)
        out_ref[...] = y.reshape(ROWS, SIZE, cout).astype(out_dtype)

    def run(x, w, a, b):
        xspec = lambda f: pl.BlockSpec((ROWS, SIZE, cin), f)
        return pl.pallas_call(
            body,
            grid=(NSTRIP,),
            in_specs=[
                xspec(lambda i: (jnp.maximum(i - 1, 0), 0, 0)),
                xspec(lambda i: (i, 0, 0)),
                xspec(lambda i: (jnp.minimum(i + 1, NSTRIP - 1), 0, 0)),
                pl.BlockSpec((3, 3, cin, cout), lambda i: (0, 0, 0, 0)),
                pl.BlockSpec((cout,), lambda i: (0,)),
                pl.BlockSpec((cout,), lambda i: (0,)),
            ],
            out_specs=pl.BlockSpec((ROWS, SIZE, cout), lambda i: (i, 0, 0)),
            out_shape=jax.ShapeDtypeStruct((SIZE, SIZE, cout), out_dtype),
        )(x, x, x, w, a, b)

    return run


def _conv1_body(prev_ref, cur_ref, nxt_ref, w_ref, a_ref, b_ref, out_ref):
    i = pl.program_id(0)
    prev = prev_ref[...].astype(jnp.bfloat16)
    cur = cur_ref[...].reshape(ROWS, SIZE, EMB).astype(jnp.bfloat16)
    nxt = nxt_ref[...].astype(jnp.bfloat16)
    top = jnp.where(i == 0, jnp.zeros((1, SIZE, EMB), jnp.bfloat16),
                    prev.reshape(1, SIZE, EMB))
    bot = jnp.where(i == NSTRIP - 1, jnp.zeros((1, SIZE, EMB), jnp.bfloat16),
                    nxt.reshape(1, SIZE, EMB))
    rows = jnp.concatenate([top, cur, bot], axis=0)
    zc = jnp.zeros((ROWS + 2, 1, EMB), jnp.bfloat16)
    strip = jnp.concatenate([zc, rows, zc], axis=1)
    acc = _conv_taps_hwc(strip, w_ref, EMB, 512)
    y = acc * a_ref[...][None, :] + b_ref[...][None, :]
    y = jnp.maximum(y, 0.0)
    out_ref[...] = y.reshape(ROWS, SIZE, 512).astype(jnp.bfloat16)


def _conv1(grid, w, a, b):
    return pl.pallas_call(
        _conv1_body,
        grid=(NSTRIP,),
        in_specs=[
            pl.BlockSpec((SIZE, EMB),
                         lambda i: (jnp.maximum(i - 1, 0) * ROWS + ROWS - 1, 0)),
            pl.BlockSpec((SPAT, EMB), lambda i: (i, 0)),
            pl.BlockSpec((SIZE, EMB),
                         lambda i: (jnp.minimum(i + 1, NSTRIP - 1) * ROWS, 0)),
            pl.BlockSpec((3, 3, EMB, 512), lambda i: (0, 0, 0, 0)),
            pl.BlockSpec((512,), lambda i: (0,)),
            pl.BlockSpec((512,), lambda i: (0,)),
        ],
        out_specs=pl.BlockSpec((ROWS, SIZE, 512), lambda i: (i, 0, 0)),
        out_shape=jax.ShapeDtypeStruct((SIZE, SIZE, 512), jnp.bfloat16),
    )(grid, grid, grid, w, a, b)


def _conv4_body(prev_ref, cur_ref, nxt_ref, w_ref, a_ref, b_ref, feat_ref,
                out_ref, y_ref, psum_ref):
    i = pl.program_id(0)
    strip = _strip_hwc(prev_ref[...], cur_ref[...], nxt_ref[...], i, 256)
    acc = _conv_taps_hwc(strip, w_ref, 256, 256)
    o = acc * a_ref[...][None, :] + b_ref[...][None, :]
    o = jnp.maximum(o, 0.0)
    o3 = o.reshape(ROWS, SIZE, 256)
    out_ref[...] = o3
    y_ref[...] = o3 + feat_ref[...]

    @pl.when(i == 0)
    def _():
        psum_ref[...] = jnp.zeros((1, 1), jnp.float32)

    psum_ref[...] += jnp.sum(o).reshape(1, 1)


def _conv4(x, w, a, b, feat):
    xspec = lambda f: pl.BlockSpec((ROWS, SIZE, 256), f)
    return pl.pallas_call(
        _conv4_body,
        grid=(NSTRIP,),
        in_specs=[
            xspec(lambda i: (jnp.maximum(i - 1, 0), 0, 0)),
            xspec(lambda i: (i, 0, 0)),
            xspec(lambda i: (jnp.minimum(i + 1, NSTRIP - 1), 0, 0)),
            pl.BlockSpec((3, 3, 256, 256), lambda i: (0, 0, 0, 0)),
            pl.BlockSpec((256,), lambda i: (0,)),
            pl.BlockSpec((256,), lambda i: (0,)),
            xspec(lambda i: (i, 0, 0)),
        ],
        out_specs=[
            pl.BlockSpec((ROWS, SIZE, 256), lambda i: (i, 0, 0)),
            pl.BlockSpec((ROWS, SIZE, 256), lambda i: (i, 0, 0)),
            pl.BlockSpec((1, 1), lambda i: (0, 0)),
        ],
        out_shape=[
            jax.ShapeDtypeStruct((SIZE, SIZE, 256), jnp.float32),
            jax.ShapeDtypeStruct((SIZE, SIZE, 256), jnp.float32),
            jax.ShapeDtypeStruct((1, 1), jnp.float32),
        ],
    )(x, x, x, w, a, b, feat)


def _ow4_body(x_ref, w_ref, b_ref, out_ref):
    xt = x_ref[...].reshape(SPAT, 256)
    acc = lax.dot_general(xt, w_ref[...], (((1,), (0,)), ((), ())),
                          preferred_element_type=jnp.float32)
    acc = acc + b_ref[...][None, :]
    out_ref[...] = acc.reshape(ROWS, SIZE, 256)


def _ow4(x, w, b):
    return pl.pallas_call(
        _ow4_body,
        grid=(NSTRIP,),
        in_specs=[
            pl.BlockSpec((ROWS, SIZE, 256), lambda i: (i, 0, 0)),
            pl.BlockSpec((256, 256), lambda i: (0, 0)),
            pl.BlockSpec((256,), lambda i: (0,)),
        ],
        out_specs=pl.BlockSpec((ROWS, SIZE, 256), lambda i: (i, 0, 0)),
        out_shape=jax.ShapeDtypeStruct((SIZE, SIZE, 256), jnp.float32),
    )(x, w, b)


def _prep_w(w):
    return jnp.transpose(w, (2, 3, 1, 0)).astype(jnp.bfloat16)


def kernel(features, target_points, visible_points, transform, all_pts_mask,
           params):
    p = params
    inv_s = 1.0 / jnp.sqrt(jnp.float32(1.0 + 1e-5))
    tp = target_points[0]
    tp8 = jnp.concatenate(
        [tp, jnp.ones((NV, 1), jnp.float32),
         jnp.zeros((NV, EMB - 4), jnp.float32)], axis=1)
    vis = visible_points[0].astype(jnp.int32)
    apm = all_pts_mask[0].astype(jnp.int32)
    filler = p['filler'].astype(jnp.float32)

    tof0, xyz8 = _sc_gather(p['emb'].astype(jnp.float32), tp8, vis)
    tofp, pcoord, pidx0, pidx1, aidx0, aidx1 = _geom(
        xyz8, transform[0], tof0, filler, apm)
    interp = _knn(apm.astype(jnp.float32), pcoord, tofp)

    fillrow = filler.reshape(1, EMB)
    fillblk = jnp.broadcast_to(filler[None, :], (128, EMB))
    grid = _sc_scatter(interp, tofp, fillrow, fillblk,
                       aidx0, aidx1, pidx0, pidx1)

    def ab(g, cb, be):
        a = g * inv_s
        return a, cb * a + be

    a1, b1 = ab(p['g1'], p['cb1'], p['be1'])
    a2, b2 = ab(p['g2'], p['cb2'], p['be2'])
    a3, b3 = ab(p['g3'], p['cb3'], p['be3'])
    a4, b4 = ab(p['g4'], p['cb4'], p['be4'])

    x1 = _conv1(grid, _prep_w(p['cw1']), a1, b1)
    x2 = _make_conv_hwc(512, 512, True, jnp.bfloat16)(
        x1, _prep_w(p['cw2']), a2, b2)
    x3 = _make_conv_hwc(512, 256, True, jnp.bfloat16)(
        x2, _prep_w(p['cw3']), a3, b3)
    feat = jnp.transpose(features[0], (1, 2, 0))
    output, y, psum = _conv4(x3, _prep_w(p['cw4']), a4, b4, feat)

    ones = jnp.ones((256,), jnp.float32)
    z1 = _make_conv_hwc(256, 256, True, jnp.bfloat16, in_f32=True)(
        y, _prep_w(p['ow1']), ones, p['ob1'])
    z2 = _make_conv_hwc(256, 256, True, jnp.bfloat16)(
        z1, _prep_w(p['ow2']), ones, p['ob2'])
    z3 = _make_conv_hwc(256, 256, True, jnp.bfloat16)(
        z2, _prep_w(p['ow3']), ones, p['ob3'])
    processed = _ow4(z3, jnp.transpose(p['ow4'][:, :, 0, 0]).astype(jnp.bfloat16),
                     p['ob4'])

    def tr(o):
        return jnp.transpose(o, (2, 0, 1))[None]

    return (tr(processed), tr(output), tr(y), psum[0, 0])


# speedup vs baseline: 1.8739x; 1.0057x over previous
import functools

import jax
import jax.numpy as jnp
from jax import lax
from jax.experimental import pallas as pl
from jax.experimental.pallas import tpu as pltpu
from jax.experimental.pallas import tpu_sc as plsc

EMB = 128
NV = 10000
KNN = 11
SIZE = 296
NPIX = SIZE * SIZE
CHUNK = 2744
GRID_ROWS = 32 * CHUNK
HALF = 16 * CHUNK
TRASH0 = NPIX
TRASH1 = NPIX + 8
NQ = 4096
NS = 2048
QBLK = 512
ROWS = 8
NSTRIP = SIZE // ROWS
SPAT = ROWS * SIZE


def _sc_gather_body(emb_hbm, tp8_hbm, vis_hbm, tof_hbm, xyz_hbm,
                    idx_v, emb_v, xyz_v, sem):
    c = lax.axis_index("c")
    s = lax.axis_index("s")
    wid = c * 16 + s
    base = wid * 64
    pltpu.sync_copy(vis_hbm.at[pl.ds(base, 64)], idx_v)
    pltpu.async_copy(emb_hbm.at[idx_v], emb_v, sem).wait()
    pltpu.sync_copy(emb_v, tof_hbm.at[pl.ds(base, 64)])
    pltpu.async_copy(tp8_hbm.at[idx_v], xyz_v, sem).wait()
    pltpu.sync_copy(xyz_v, xyz_hbm.at[pl.ds(base, 64)])


def _sc_gather(emb, tp8, vis):
    mesh = plsc.VectorSubcoreMesh(core_axis_name="c", subcore_axis_name="s")
    kern = functools.partial(
        pl.kernel,
        mesh=mesh,
        compiler_params=pltpu.CompilerParams(use_tc_tiling_on_sc=True),
        out_type=[
            jax.ShapeDtypeStruct((NS, EMB), jnp.float32),
            jax.ShapeDtypeStruct((NS, EMB), jnp.float32),
        ],
        scratch_types=[
            pltpu.VMEM((64,), jnp.int32),
            pltpu.VMEM((64, EMB), jnp.float32),
            pltpu.VMEM((64, EMB), jnp.float32),
            pltpu.SemaphoreType.DMA,
        ],
    )(_sc_gather_body)
    return kern(emb, tp8, vis)


def _sc_scatter_body(interp_hbm, tofp_hbm, fillrow_hbm, fillblk_hbm,
                     aidx0_hbm, aidx1_hbm, pidx0_hbm, pidx1_hbm, grid_hbm,
                     fill_v, idxa_v, rowsa_v, idxp_v, rowsp_v, sem):
    c = lax.axis_index("c")
    s = lax.axis_index("s")
    pltpu.sync_copy(fillblk_hbm, fill_v.at[pl.ds(0, 128)])
    pltpu.sync_copy(fillblk_hbm, fill_v.at[pl.ds(128, 128)])
    fbase = (c * 16 + s) * CHUNK
    copies = []
    for j in range(CHUNK // 256):
        copies.append(pltpu.async_copy(
            fill_v, grid_hbm.at[pl.ds(fbase + j * 256, 256)], sem))
    rem = CHUNK - 256 * (CHUNK // 256)
    copies.append(pltpu.async_copy(
        fill_v.at[pl.ds(0, rem)],
        grid_hbm.at[pl.ds(fbase + 256 * (CHUNK // 256), rem)], sem))
    for cp in copies:
        cp.wait()
    plsc.subcore_barrier()
    abase = s * 256

    @pl.when(c == 0)
    def _():
        for j in range(2):
            pltpu.sync_copy(aidx0_hbm.at[pl.ds(abase + j * 128, 128)],
                            idxa_v.at[j])

    @pl.when(c == 1)
    def _():
        for j in range(2):
            pltpu.sync_copy(aidx1_hbm.at[pl.ds(abase + j * 128, 128)],
                            idxa_v.at[j])

    for j in range(2):
        pltpu.sync_copy(interp_hbm.at[pl.ds(abase + j * 128, 128)],
                        rowsa_v.at[j])
        pltpu.async_copy(rowsa_v.at[j], grid_hbm.at[idxa_v.at[j]], sem).wait()
    plsc.subcore_barrier()
    pbase = s * 128

    @pl.when(c == 0)
    def _():
        pltpu.sync_copy(pidx0_hbm.at[pl.ds(pbase, 128)], idxp_v)

    @pl.when(c == 1)
    def _():
        pltpu.sync_copy(pidx1_hbm.at[pl.ds(pbase, 128)], idxp_v)

    pltpu.sync_copy(tofp_hbm.at[pl.ds(pbase, 128)], rowsp_v)
    pltpu.async_copy(rowsp_v, grid_hbm.at[idxp_v], sem).wait()
    plsc.subcore_barrier()
    @pl.when((c == 0) & (s == 0))
    def _():
        pltpu.sync_copy(fillrow_hbm, grid_hbm.at[pl.ds(0, 1)])


def _sc_scatter(interp, tofp, fillrow, fillblk, aidx0, aidx1, pidx0, pidx1):
    mesh = plsc.VectorSubcoreMesh(core_axis_name="c", subcore_axis_name="s")
    kern = functools.partial(
        pl.kernel,
        mesh=mesh,
        compiler_params=pltpu.CompilerParams(use_tc_tiling_on_sc=True),
        out_type=jax.ShapeDtypeStruct((GRID_ROWS, EMB), jnp.float32),
        scratch_types=[
            pltpu.VMEM((256, EMB), jnp.float32),
            pltpu.VMEM((2, 128), jnp.int32),
            pltpu.VMEM((2, 128, EMB), jnp.float32),
            pltpu.VMEM((128,), jnp.int32),
            pltpu.VMEM((128, EMB), jnp.float32),
            pltpu.SemaphoreType.DMA,
        ],
    )(_sc_scatter_body)
    return kern(interp, tofp, fillrow, fillblk, aidx0, aidx1, pidx0, pidx1)


def _geom_body(xyz_ref, tr_ref, tof0_ref, fill_ref, apm_ref,
               tofp_ref, pcoord_ref, pidx0_ref, pidx1_ref,
               aidx0_ref, aidx1_ref):
    ph = xyz_ref[:, 0:4]
    tr = tr_ref[...]
    cam = lax.dot_general(ph, tr, (((1,), (1,)), ((), ())),
                          preferred_element_type=jnp.float32)
    x = cam[:, 0]
    y = cam[:, 1]
    z = cam[:, 2]
    zs = jnp.where(jnp.abs(z) < 1e-6, 1e-6, z)
    u = 12.0 * x / zs + SIZE / 2.0
    v = 12.0 * y / zs + SIZE / 2.0
    px = jnp.round(u)
    py = jnp.round(v)
    bad = (px < 0.0) | (px > 295.0) | (py < 0.0) | (py > 295.0)
    px = jnp.where(bad, 0.0, px)
    py = jnp.where(bad, 0.0, py)
    tofp_ref[...] = jnp.where(bad[:, None], fill_ref[...][None, :],
                              tof0_ref[...])
    pcoord_ref[0:1, :] = px.reshape(1, NS)
    pcoord_ref[1:2, :] = py.reshape(1, NS)
    pcoord_ref[2:8, :] = jnp.zeros((6, NS), jnp.float32)
    pid = py.astype(jnp.int32) * SIZE + px.astype(jnp.int32)
    pid_row = pid.reshape(1, NS)
    col = lax.broadcasted_iota(jnp.int32, (1, NS), 1)
    loser_parts = []
    for cblk in range(4):
        pc_chunk = pid[cblk * 512:(cblk + 1) * 512]
        row_ids = lax.broadcasted_iota(jnp.int32, (512, 1), 0) + cblk * 512
        eq = (pc_chunk[:, None] == pid_row) & (col > row_ids)
        loser_parts.append(jnp.any(eq, axis=1).astype(jnp.int32))
    loser = jnp.concatenate(loser_parts, axis=0)
    own0 = pid < HALF
    keep = loser == 0
    pidx0_ref[...] = jnp.where(own0 & keep, pid, TRASH0)
    pidx1_ref[...] = jnp.where(jnp.logical_not(own0) & keep, pid, TRASH1)
    apm = apm_ref[...]
    aid = apm[:, 1] * SIZE + apm[:, 0]
    aown0 = aid < HALF
    aidx0_ref[...] = jnp.where(aown0, aid, TRASH0)
    aidx1_ref[...] = jnp.where(jnp.logical_not(aown0), aid, TRASH1)


def _geom(xyz8, transform, tof0, filler, apm):
    return pl.pallas_call(
        _geom_body,
        out_shape=[
            jax.ShapeDtypeStruct((NS, EMB), jnp.float32),
            jax.ShapeDtypeStruct((8, NS), jnp.float32),
            jax.ShapeDtypeStruct((NS,), jnp.int32),
            jax.ShapeDtypeStruct((NS,), jnp.int32),
            jax.ShapeDtypeStruct((NQ,), jnp.int32),
            jax.ShapeDtypeStruct((NQ,), jnp.int32),
        ],
    )(xyz8, transform, tof0, filler, apm)


def _knn_body(apmf_ref, pcoord_ref, tofp_ref, out_ref):
    qx = apmf_ref[:, 0:1]
    qy = apmf_ref[:, 1:2]
    px = pcoord_ref[0:1, :]
    py = pcoord_ref[1:2, :]
    dx = qx - px
    dy = qy - py
    d2 = dx * dx + dy * dy
    col = lax.broadcasted_iota(jnp.int32, (QBLK, NS), 1)
    key = d2.astype(jnp.int32) * NS + col
    big = jnp.int32(2147483647)
    kw = key
    for _ in range(KNN - 1):
        m = jnp.min(kw, axis=1, keepdims=True)
        kw = jnp.where(kw == m, big, kw)
    t11 = jnp.min(kw, axis=1, keepdims=True)
    wt = 1.0 / (jnp.sqrt(d2 + 1e-12) + 1e-8)
    wmat = jnp.where(key <= t11, wt, 0.0)
    wsum = jnp.sum(wmat, axis=1, keepdims=True)
    wmat = wmat / wsum
    out_ref[...] = jnp.dot(wmat, tofp_ref[...],
                           preferred_element_type=jnp.float32)


def _knn(apmf, pcoord, tofp):
    return pl.pallas_call(
        _knn_body,
        grid=(NQ // QBLK,),
        in_specs=[
            pl.BlockSpec((QBLK, 2), lambda i: (i, 0)),
            pl.BlockSpec((8, NS), lambda i: (0, 0)),
            pl.BlockSpec((NS, EMB), lambda i: (0, 0)),
        ],
        out_specs=pl.BlockSpec((QBLK, EMB), lambda i: (i, 0)),
        out_shape=jax.ShapeDtypeStruct((NQ, EMB), jnp.float32),
    )(apmf, pcoord, tofp)


def _strip_hwc(prev, cur, nxt, i, cin):
    top = jnp.where(i == 0, jnp.zeros((1, SIZE, cin), jnp.bfloat16), prev)
    bot = jnp.where(i == NSTRIP - 1, jnp.zeros((1, SIZE, cin), jnp.bfloat16),
                    nxt)
    rows = jnp.concatenate([top, cur, bot], axis=0)
    zc = jnp.zeros((ROWS + 2, 1, cin), jnp.bfloat16)
    return jnp.concatenate([zc, rows, zc], axis=1)


def _conv_taps_hwc(strip, w_ref, cin, cout):
    acc = jnp.zeros((SPAT, cout), jnp.float32)
    for kw in range(3):
        xs = strip[:, kw:kw + SIZE, :]
        for kh in range(3):
            xt = xs[kh:kh + ROWS].reshape(SPAT, cin)
            acc = acc + lax.dot_general(
                xt, w_ref[kh, kw], (((1,), (0,)), ((), ())),
                preferred_element_type=jnp.float32)
    return acc


def _make_conv_hwc(cin, cout, relu, out_dtype, in_f32=False):
    def body(prev_ref, cur_ref, nxt_ref, w_ref, a_ref, b_ref, out_ref):
        i = pl.program_id(0)
        if in_f32:
            prev = prev_ref[...].astype(jnp.bfloat16)
            cur = cur_ref[...].astype(jnp.bfloat16)
            nxt = nxt_ref[...].astype(jnp.bfloat16)
        else:
            prev, cur, nxt = prev_ref[...], cur_ref[...], nxt_ref[...]
        strip = _strip_hwc(prev, cur, nxt, i, cin)
        acc = _conv_taps_hwc(strip, w_ref, cin, cout)
        y = acc * a_ref[...][None, :] + b_ref[...][None, :]
        if relu:
            y = jnp.maximum(y, 0.0)
        out_ref[...] = y.reshape(ROWS, SIZE, cout).astype(out_dtype)

    def run(x, w, a, b):
        hspec = lambda f: pl.BlockSpec((1, SIZE, cin), f)
        return pl.pallas_call(
            body,
            grid=(NSTRIP,),
            in_specs=[
                hspec(lambda i: (jnp.maximum(i * ROWS - 1, 0), 0, 0)),
                pl.BlockSpec((ROWS, SIZE, cin), lambda i: (i, 0, 0)),
                hspec(lambda i: (jnp.minimum(i * ROWS + ROWS, SIZE - 1), 0, 0)),
                pl.BlockSpec((3, 3, cin, cout), lambda i: (0, 0, 0, 0)),
                pl.BlockSpec((cout,), lambda i: (0,)),
                pl.BlockSpec((cout,), lambda i: (0,)),
            ],
            out_specs=pl.BlockSpec((ROWS, SIZE, cout), lambda i: (i, 0, 0)),
            out_shape=jax.ShapeDtypeStruct((SIZE, SIZE, cout), out_dtype),
        )(x, x, x, w, a, b)

    return run


def _conv1_body(prev_ref, cur_ref, nxt_ref, w_ref, a_ref, b_ref, out_ref):
    i = pl.program_id(0)
    prev = prev_ref[...].astype(jnp.bfloat16)
    cur = cur_ref[...].reshape(ROWS, SIZE, EMB).astype(jnp.bfloat16)
    nxt = nxt_ref[...].astype(jnp.bfloat16)
    top = jnp.where(i == 0, jnp.zeros((1, SIZE, EMB), jnp.bfloat16),
                    prev.reshape(1, SIZE, EMB))
    bot = jnp.where(i == NSTRIP - 1, jnp.zeros((1, SIZE, EMB), jnp.bfloat16),
                    nxt.reshape(1, SIZE, EMB))
    rows = jnp.concatenate([top, cur, bot], axis=0)
    zc = jnp.zeros((ROWS + 2, 1, EMB), jnp.bfloat16)
    strip = jnp.concatenate([zc, rows, zc], axis=1)
    acc = _conv_taps_hwc(strip, w_ref, EMB, 512)
    y = acc * a_ref[...][None, :] + b_ref[...][None, :]
    y = jnp.maximum(y, 0.0)
    out_ref[...] = y.reshape(ROWS, SIZE, 512).astype(jnp.bfloat16)


def _conv1(grid, w, a, b):
    return pl.pallas_call(
        _conv1_body,
        grid=(NSTRIP,),
        in_specs=[
            pl.BlockSpec((SIZE, EMB),
                         lambda i: (jnp.maximum(i - 1, 0) * ROWS + ROWS - 1, 0)),
            pl.BlockSpec((SPAT, EMB), lambda i: (i, 0)),
            pl.BlockSpec((SIZE, EMB),
                         lambda i: (jnp.minimum(i + 1, NSTRIP - 1) * ROWS, 0)),
            pl.BlockSpec((3, 3, EMB, 512), lambda i: (0, 0, 0, 0)),
            pl.BlockSpec((512,), lambda i: (0,)),
            pl.BlockSpec((512,), lambda i: (0,)),
        ],
        out_specs=pl.BlockSpec((ROWS, SIZE, 512), lambda i: (i, 0, 0)),
        out_shape=jax.ShapeDtypeStruct((SIZE, SIZE, 512), jnp.bfloat16),
    )(grid, grid, grid, w, a, b)


def _conv4_body(prev_ref, cur_ref, nxt_ref, w_ref, a_ref, b_ref, feat_ref,
                out_ref, y_ref, psum_ref):
    i = pl.program_id(0)
    strip = _strip_hwc(prev_ref[...], cur_ref[...], nxt_ref[...], i, 256)
    acc = _conv_taps_hwc(strip, w_ref, 256, 256)
    o = acc * a_ref[...][None, :] + b_ref[...][None, :]
    o = jnp.maximum(o, 0.0)
    o3 = o.reshape(ROWS, SIZE, 256)
    out_ref[...] = o3
    y_ref[...] = o3 + feat_ref[...]

    @pl.when(i == 0)
    def _():
        psum_ref[...] = jnp.zeros((1, 1), jnp.float32)

    psum_ref[...] += jnp.sum(o).reshape(1, 1)


def _conv4(x, w, a, b, feat):
    xspec = lambda f: pl.BlockSpec((ROWS, SIZE, 256), f)
    hspec = lambda f: pl.BlockSpec((1, SIZE, 256), f)
    return pl.pallas_call(
        _conv4_body,
        grid=(NSTRIP,),
        in_specs=[
            hspec(lambda i: (jnp.maximum(i * ROWS - 1, 0), 0, 0)),
            xspec(lambda i: (i, 0, 0)),
            hspec(lambda i: (jnp.minimum(i * ROWS + ROWS, SIZE - 1), 0, 0)),
            pl.BlockSpec((3, 3, 256, 256), lambda i: (0, 0, 0, 0)),
            pl.BlockSpec((256,), lambda i: (0,)),
            pl.BlockSpec((256,), lambda i: (0,)),
            xspec(lambda i: (i, 0, 0)),
        ],
        out_specs=[
            pl.BlockSpec((ROWS, SIZE, 256), lambda i: (i, 0, 0)),
            pl.BlockSpec((ROWS, SIZE, 256), lambda i: (i, 0, 0)),
            pl.BlockSpec((1, 1), lambda i: (0, 0)),
        ],
        out_shape=[
            jax.ShapeDtypeStruct((SIZE, SIZE, 256), jnp.float32),
            jax.ShapeDtypeStruct((SIZE, SIZE, 256), jnp.float32),
            jax.ShapeDtypeStruct((1, 1), jnp.float32),
        ],
    )(x, x, x, w, a, b, feat)


def _ow4_body(x_ref, w_ref, b_ref, out_ref):
    xt = x_ref[...].reshape(SPAT, 256)
    acc = lax.dot_general(xt, w_ref[...], (((1,), (0,)), ((), ())),
                          preferred_element_type=jnp.float32)
    acc = acc + b_ref[...][None, :]
    out_ref[...] = acc.reshape(ROWS, SIZE, 256)


def _ow4(x, w, b):
    return pl.pallas_call(
        _ow4_body,
        grid=(NSTRIP,),
        in_specs=[
            pl.BlockSpec((ROWS, SIZE, 256), lambda i: (i, 0, 0)),
            pl.BlockSpec((256, 256), lambda i: (0, 0)),
            pl.BlockSpec((256,), lambda i: (0,)),
        ],
        out_specs=pl.BlockSpec((ROWS, SIZE, 256), lambda i: (i, 0, 0)),
        out_shape=jax.ShapeDtypeStruct((SIZE, SIZE, 256), jnp.float32),
    )(x, w, b)


def _prep_w(w):
    return jnp.transpose(w, (2, 3, 1, 0)).astype(jnp.bfloat16)


def kernel(features, target_points, visible_points, transform, all_pts_mask,
           params):
    p = params
    inv_s = 1.0 / jnp.sqrt(jnp.float32(1.0 + 1e-5))
    tp = target_points[0]
    tp8 = jnp.concatenate(
        [tp, jnp.ones((NV, 1), jnp.float32),
         jnp.zeros((NV, EMB - 4), jnp.float32)], axis=1)
    vis = visible_points[0].astype(jnp.int32)
    apm = all_pts_mask[0].astype(jnp.int32)
    filler = p['filler'].astype(jnp.float32)

    tof0, xyz8 = _sc_gather(p['emb'].astype(jnp.float32), tp8, vis)
    tofp, pcoord, pidx0, pidx1, aidx0, aidx1 = _geom(
        xyz8, transform[0], tof0, filler, apm)
    interp = _knn(apm.astype(jnp.float32), pcoord, tofp)

    fillrow = filler.reshape(1, EMB)
    fillblk = jnp.broadcast_to(filler[None, :], (128, EMB))
    grid = _sc_scatter(interp, tofp, fillrow, fillblk,
                       aidx0, aidx1, pidx0, pidx1)

    def ab(g, cb, be):
        a = g * inv_s
        return a, cb * a + be

    a1, b1 = ab(p['g1'], p['cb1'], p['be1'])
    a2, b2 = ab(p['g2'], p['cb2'], p['be2'])
    a3, b3 = ab(p['g3'], p['cb3'], p['be3'])
    a4, b4 = ab(p['g4'], p['cb4'], p['be4'])

    x1 = _conv1(grid, _prep_w(p['cw1']), a1, b1)
    x2 = _make_conv_hwc(512, 512, True, jnp.bfloat16)(
        x1, _prep_w(p['cw2']), a2, b2)
    x3 = _make_conv_hwc(512, 256, True, jnp.bfloat16)(
        x2, _prep_w(p['cw3']), a3, b3)
    feat = jnp.transpose(features[0], (1, 2, 0))
    output, y, psum = _conv4(x3, _prep_w(p['cw4']), a4, b4, feat)

    ones = jnp.ones((256,), jnp.float32)
    z1 = _make_conv_hwc(256, 256, True, jnp.bfloat16, in_f32=True)(
        y, _prep_w(p['ow1']), ones, p['ob1'])
    z2 = _make_conv_hwc(256, 256, True, jnp.bfloat16)(
        z1, _prep_w(p['ow2']), ones, p['ob2'])
    z3 = _make_conv_hwc(256, 256, True, jnp.bfloat16)(
        z2, _prep_w(p['ow3']), ones, p['ob3'])
    processed = _ow4(z3, jnp.transpose(p['ow4'][:, :, 0, 0]).astype(jnp.bfloat16),
                     p['ob4'])

    def tr(o):
        return jnp.transpose(o, (2, 0, 1))[None]

    return (tr(processed), tr(output), tr(y), psum[0, 0])
